# baseline (device time: 39907 ns/iter reference)
import jax
import jax.numpy as jnp
from jax import lax
from jax.experimental import pallas as pl
from jax.experimental.pallas import tpu as pltpu

N_DEV = 16
B = 2
SQ = 128
HQ = 8
HKV = 2
DH = 64
D = HQ * DH
SKV = 2048
SKV_PER = SKV // N_DEV
GROUP = HQ // HKV
SCALE = 0.125
ROWS = B * SQ
RPD = ROWS // N_DEV


def kernel(x, Wq, Wo, K_ext, V_ext):
    def body(
        x_ref,
        wq_ref,
        wo_ref,
        k_ref,
        v_ref,
        out_ref,
        obuf,
        sbuf,
        orecv,
        srecv,
        rs_o_send,
        rs_o_recv,
        rs_s_send,
        rs_s_recv,
        ag_send,
        ag_recv,
    ):
        my = lax.axis_index("i")
        my_b = lax.div(my, N_DEV // B)
        my_sq0 = lax.rem(my, N_DEV // B) * RPD

        barrier_sem = pltpu.get_barrier_semaphore()
        for d in range(1, N_DEV):
            nbr = lax.rem(my + d, N_DEV)
            pl.semaphore_signal(
                barrier_sem,
                inc=1,
                device_id=(nbr,),
                device_id_type=pl.DeviceIdType.MESH,
            )
        pl.semaphore_wait(barrier_sem, N_DEV - 1)

        wq = wq_ref[...].astype(jnp.bfloat16)
        wo = wo_ref[...].astype(jnp.bfloat16)
        for b in range(B):
            xb = x_ref[b, :, :].astype(jnp.bfloat16)
            qb = jnp.dot(xb, wq, preferred_element_type=jnp.float32)
            for h in range(HQ):
                hkv = h // GROUP
                qh = qb[:, h * DH : (h + 1) * DH].astype(jnp.bfloat16)
                kb = k_ref[b, :, hkv, :].astype(jnp.bfloat16)
                vb = v_ref[b, :, hkv, :].astype(jnp.bfloat16)
                s = (
                    jnp.dot(qh, kb.T, preferred_element_type=jnp.float32)
                    * SCALE
                )
                m = jnp.max(s, axis=1, keepdims=True)
                p = jnp.exp(s - m)
                l = jnp.sum(p, axis=1, keepdims=True)
                o = jnp.dot(
                    p.astype(jnp.bfloat16), vb, preferred_element_type=jnp.float32
                )
                obuf[0, h, b * SQ : (b + 1) * SQ, :] = o
                sbuf[0, 0, b * SQ : (b + 1) * SQ, h : h + 1] = m
                sbuf[0, 1, b * SQ : (b + 1) * SQ, h : h + 1] = l

        orecv[pl.ds(my, 1)] = obuf[:, :, pl.ds(my * RPD, RPD), :]
        srecv[pl.ds(my, 1)] = sbuf[:, :, pl.ds(my * RPD, RPD), :]

        sends = []
        recvs = []
        for d in range(1, N_DEV):
            t = lax.rem(my + d, N_DEV)
            j = N_DEV - 1 - d
            rd_o = pltpu.make_async_remote_copy(
                src_ref=obuf.at[:, :, pl.ds(t * RPD, RPD), :],
                dst_ref=orecv.at[pl.ds(my, 1)],
                send_sem=rs_o_send.at[d - 1],
                recv_sem=rs_o_recv.at[j],
                device_id=(t,),
                device_id_type=pl.DeviceIdType.MESH,
            )
            rd_s = pltpu.make_async_remote_copy(
                src_ref=sbuf.at[:, :, pl.ds(t * RPD, RPD), :],
                dst_ref=srecv.at[pl.ds(my, 1)],
                send_sem=rs_s_send.at[d - 1],
                recv_sem=rs_s_recv.at[j],
                device_id=(t,),
                device_id_type=pl.DeviceIdType.MESH,
            )
            rd_o.start()
            rd_s.start()
            sends += [rd_o, rd_s]
            s_idx = lax.rem(my - d + N_DEV, N_DEV)
            recvs.append(
                pltpu.make_async_remote_copy(
                    src_ref=orecv.at[pl.ds(s_idx, 1)],
                    dst_ref=orecv.at[pl.ds(s_idx, 1)],
                    send_sem=rs_o_send.at[d - 1],
                    recv_sem=rs_o_recv.at[j],
                    device_id=(s_idx,),
                    device_id_type=pl.DeviceIdType.MESH,
                )
            )
            recvs.append(
                pltpu.make_async_remote_copy(
                    src_ref=srecv.at[pl.ds(s_idx, 1)],
                    dst_ref=srecv.at[pl.ds(s_idx, 1)],
                    send_sem=rs_s_send.at[d - 1],
                    recv_sem=rs_s_recv.at[j],
                    device_id=(s_idx,),
                    device_id_type=pl.DeviceIdType.MESH,
                )
            )
        for rdma in recvs:
            rdma.wait_recv()

        acc = jnp.zeros((RPD, D), dtype=jnp.float32)
        for h in range(HQ):
            m_all = srecv[:, 0, :, h]
            l_all = srecv[:, 1, :, h]
            m_h = jnp.max(m_all, axis=0, keepdims=True)
            alpha = jnp.exp(m_all - m_h)
            l_h = jnp.sum(l_all * alpha, axis=0, keepdims=True)
            o_all = orecv[:, h, :, :]
            o_num = jnp.sum(o_all * alpha[:, :, None], axis=0)
            attn_h = o_num * jnp.transpose(1.0 / l_h)
            acc = acc + jnp.dot(
                attn_h.astype(jnp.bfloat16),
                wo[h * DH : (h + 1) * DH, :],
                preferred_element_type=jnp.float32,
            )
        out_ref[pl.ds(my_b, 1), pl.ds(my_sq0, RPD), :] = acc[None]

        my_out = out_ref.at[pl.ds(my_b, 1), pl.ds(my_sq0, RPD), :]
        ag_recvs = []
        for d in range(1, N_DEV):
            t = lax.rem(my + d, N_DEV)
            j = N_DEV - 1 - d
            rd = pltpu.make_async_remote_copy(
                src_ref=my_out,
                dst_ref=my_out,
                send_sem=ag_send.at[d - 1],
                recv_sem=ag_recv.at[j],
                device_id=(t,),
                device_id_type=pl.DeviceIdType.MESH,
            )
            rd.start()
            sends.append(rd)
            s_idx = lax.rem(my - d + N_DEV, N_DEV)
            s_b = lax.div(s_idx, N_DEV // B)
            s_sq0 = lax.rem(s_idx, N_DEV // B) * RPD
            s_out = out_ref.at[pl.ds(s_b, 1), pl.ds(s_sq0, RPD), :]
            ag_recvs.append(
                pltpu.make_async_remote_copy(
                    src_ref=s_out,
                    dst_ref=s_out,
                    send_sem=ag_send.at[d - 1],
                    recv_sem=ag_recv.at[j],
                    device_id=(s_idx,),
                    device_id_type=pl.DeviceIdType.MESH,
                )
            )
        for rdma in ag_recvs:
            rdma.wait_recv()
        for rdma in sends:
            rdma.wait_send()

    return pl.pallas_call(
        body,
        out_shape=jax.ShapeDtypeStruct((B, SQ, D), jnp.float32),
        in_specs=[pl.BlockSpec(memory_space=pltpu.VMEM)] * 5,
        out_specs=pl.BlockSpec(memory_space=pltpu.VMEM),
        scratch_shapes=[
            pltpu.VMEM((1, HQ, ROWS, DH), jnp.float32),
            pltpu.VMEM((1, 2, ROWS, HQ), jnp.float32),
            pltpu.VMEM((N_DEV, HQ, RPD, DH), jnp.float32),
            pltpu.VMEM((N_DEV, 2, RPD, HQ), jnp.float32),
            pltpu.SemaphoreType.DMA((N_DEV - 1,)),
            pltpu.SemaphoreType.DMA((N_DEV - 1,)),
            pltpu.SemaphoreType.DMA((N_DEV - 1,)),
            pltpu.SemaphoreType.DMA((N_DEV - 1,)),
            pltpu.SemaphoreType.DMA((N_DEV - 1,)),
            pltpu.SemaphoreType.DMA((N_DEV - 1,)),
        ],
        compiler_params=pltpu.CompilerParams(collective_id=0),
    )(x, Wq, Wo, K_ext, V_ext)


# device time: 32282 ns/iter; 1.2362x vs baseline; 1.2362x over previous
import jax
import jax.numpy as jnp
from jax import lax
from jax.experimental import pallas as pl
from jax.experimental.pallas import tpu as pltpu

N_DEV = 16
B = 2
SQ = 128
HQ = 8
HKV = 2
DH = 64
D = HQ * DH
SKV = 2048
SKV_PER = SKV // N_DEV
GROUP = HQ // HKV
SCALE = 0.125
ROWS = B * SQ
RPD = ROWS // N_DEV
LANES = 128


def kernel(x, Wq, Wo, K_ext, V_ext):
    def body(
        x_ref,
        wq_ref,
        wo_ref,
        k_ref,
        v_ref,
        out_ref,
        obuf,
        orecv,
        rs_send,
        rs_recv,
        ag_send,
        ag_recv,
    ):
        my = lax.axis_index("i")
        my_b = lax.div(my, N_DEV // B)
        my_sq0 = lax.rem(my, N_DEV // B) * RPD

        barrier_sem = pltpu.get_barrier_semaphore()
        for d in range(1, N_DEV):
            nbr = lax.rem(my + d, N_DEV)
            pl.semaphore_signal(
                barrier_sem,
                inc=1,
                device_id=(nbr,),
                device_id_type=pl.DeviceIdType.MESH,
            )

        wq = wq_ref[...].astype(jnp.bfloat16)
        wo = wo_ref[...].astype(jnp.bfloat16)
        for b in range(B):
            xb = x_ref[b, :, :].astype(jnp.bfloat16)
            qb = jnp.dot(xb, wq, preferred_element_type=jnp.float32)
            for hkv in range(HKV):
                qcat = jnp.concatenate(
                    [
                        qb[:, (hkv * GROUP + j) * DH : (hkv * GROUP + j + 1) * DH]
                        for j in range(GROUP)
                    ],
                    axis=0,
                ).astype(jnp.bfloat16)
                kb = k_ref[b, :, hkv, :].astype(jnp.bfloat16)
                vb = v_ref[b, :, hkv, :].astype(jnp.bfloat16)
                s = (
                    jnp.dot(qcat, kb.T, preferred_element_type=jnp.float32)
                    * SCALE
                )
                m = jnp.max(s, axis=1, keepdims=True)
                p = jnp.exp(s - m)
                l = jnp.sum(p, axis=1, keepdims=True)
                o = jnp.dot(
                    p.astype(jnp.bfloat16), vb, preferred_element_type=jnp.float32
                )
                ml = jnp.concatenate([m, l], axis=1)
                for j in range(GROUP):
                    hg = hkv * GROUP + j
                    rows = slice(j * SQ, (j + 1) * SQ)
                    obuf[0, hg, b * SQ : (b + 1) * SQ, 0:DH] = o[rows]
                    obuf[0, hg, b * SQ : (b + 1) * SQ, DH : DH + 2] = ml[rows]

        orecv[pl.ds(my, 1)] = obuf[:, :, pl.ds(my * RPD, RPD), :]

        pl.semaphore_wait(barrier_sem, N_DEV - 1)

        sends = []
        recvs = []
        for d in range(1, N_DEV):
            t = lax.rem(my + d, N_DEV)
            j = N_DEV - 1 - d
            rd = pltpu.make_async_remote_copy(
                src_ref=obuf.at[:, :, pl.ds(t * RPD, RPD), :],
                dst_ref=orecv.at[pl.ds(my, 1)],
                send_sem=rs_send.at[d - 1],
                recv_sem=rs_recv.at[j],
                device_id=(t,),
                device_id_type=pl.DeviceIdType.MESH,
            )
            rd.start()
            sends.append(rd)
            s_idx = lax.rem(my - d + N_DEV, N_DEV)
            recvs.append(
                pltpu.make_async_remote_copy(
                    src_ref=orecv.at[pl.ds(s_idx, 1)],
                    dst_ref=orecv.at[pl.ds(s_idx, 1)],
                    send_sem=rs_send.at[d - 1],
                    recv_sem=rs_recv.at[j],
                    device_id=(s_idx,),
                    device_id_type=pl.DeviceIdType.MESH,
                )
            )
        for rdma in recvs:
            rdma.wait_recv()

        m_all = orecv[:, :, :, DH : DH + 1]
        l_all = orecv[:, :, :, DH + 1 : DH + 2]
        o_all = orecv[:, :, :, 0:DH]
        m_x = jnp.max(m_all, axis=0, keepdims=True)
        alpha = jnp.exp(m_all - m_x)
        l_x = jnp.sum(l_all * alpha, axis=0)
        o_num = jnp.sum(o_all * alpha, axis=0)
        attn = o_num / l_x
        attn_rows = jnp.concatenate(
            [attn[h] for h in range(HQ)], axis=1
        ).astype(jnp.bfloat16)
        acc = jnp.dot(attn_rows, wo, preferred_element_type=jnp.float32)
        out_ref[pl.ds(my_b, 1), pl.ds(my_sq0, RPD), :] = acc[None]

        my_out = out_ref.at[pl.ds(my_b, 1), pl.ds(my_sq0, RPD), :]
        ag_recvs = []
        for d in range(1, N_DEV):
            t = lax.rem(my + d, N_DEV)
            j = N_DEV - 1 - d
            rd = pltpu.make_async_remote_copy(
                src_ref=my_out,
                dst_ref=my_out,
                send_sem=ag_send.at[d - 1],
                recv_sem=ag_recv.at[j],
                device_id=(t,),
                device_id_type=pl.DeviceIdType.MESH,
            )
            rd.start()
            sends.append(rd)
            s_idx = lax.rem(my - d + N_DEV, N_DEV)
            s_b = lax.div(s_idx, N_DEV // B)
            s_sq0 = lax.rem(s_idx, N_DEV // B) * RPD
            s_out = out_ref.at[pl.ds(s_b, 1), pl.ds(s_sq0, RPD), :]
            ag_recvs.append(
                pltpu.make_async_remote_copy(
                    src_ref=s_out,
                    dst_ref=s_out,
                    send_sem=ag_send.at[d - 1],
                    recv_sem=ag_recv.at[j],
                    device_id=(s_idx,),
                    device_id_type=pl.DeviceIdType.MESH,
                )
            )
        for rdma in ag_recvs:
            rdma.wait_recv()
        for rdma in sends:
            rdma.wait_send()

    return pl.pallas_call(
        body,
        out_shape=jax.ShapeDtypeStruct((B, SQ, D), jnp.float32),
        in_specs=[pl.BlockSpec(memory_space=pltpu.VMEM)] * 5,
        out_specs=pl.BlockSpec(memory_space=pltpu.VMEM),
        scratch_shapes=[
            pltpu.VMEM((1, HQ, ROWS, LANES), jnp.float32),
            pltpu.VMEM((N_DEV, HQ, RPD, LANES), jnp.float32),
            pltpu.SemaphoreType.DMA((N_DEV - 1,)),
            pltpu.SemaphoreType.DMA((N_DEV - 1,)),
            pltpu.SemaphoreType.DMA((N_DEV - 1,)),
            pltpu.SemaphoreType.DMA((N_DEV - 1,)),
        ],
        compiler_params=pltpu.CompilerParams(collective_id=0),
    )(x, Wq, Wo, K_ext, V_ext)


# device time: 24353 ns/iter; 1.6387x vs baseline; 1.3256x over previous
import jax
import jax.numpy as jnp
from jax import lax
from jax.experimental import pallas as pl
from jax.experimental.pallas import tpu as pltpu

N_DEV = 16
B = 2
SQ = 128
HQ = 8
HKV = 2
DH = 64
D = HQ * DH
SKV = 2048
SKV_PER = SKV // N_DEV
GROUP = HQ // HKV
SCALE = 0.125
ROWS = B * SQ
RPD = ROWS // N_DEV
LANES = 128


def kernel(x, Wq, Wo, K_ext, V_ext):
    def body(
        x_ref,
        wq_ref,
        wo_ref,
        k_ref,
        v_ref,
        out_ref,
        obuf,
        orecv,
        agbuf,
        rs_send,
        rs_recv,
        ag_send,
        ag_recv,
    ):
        my = lax.axis_index("i")

        barrier_sem = pltpu.get_barrier_semaphore()
        for d in range(1, N_DEV):
            nbr = lax.rem(my + d, N_DEV)
            pl.semaphore_signal(
                barrier_sem,
                inc=1,
                device_id=(nbr,),
                device_id_type=pl.DeviceIdType.MESH,
            )

        wq = wq_ref[...].astype(jnp.bfloat16)
        wo = wo_ref[...].astype(jnp.bfloat16)
        for b in range(B):
            xb = x_ref[b, :, :].astype(jnp.bfloat16)
            qb = jnp.dot(xb, wq, preferred_element_type=jnp.float32)
            for hkv in range(HKV):
                qcat = jnp.concatenate(
                    [
                        qb[:, (hkv * GROUP + j) * DH : (hkv * GROUP + j + 1) * DH]
                        for j in range(GROUP)
                    ],
                    axis=0,
                ).astype(jnp.bfloat16)
                kb = k_ref[b, :, hkv, :].astype(jnp.bfloat16)
                vb = v_ref[b, :, hkv, :].astype(jnp.bfloat16)
                s = (
                    jnp.dot(qcat, kb.T, preferred_element_type=jnp.float32)
                    * SCALE
                )
                m = jnp.max(s, axis=1, keepdims=True)
                m_r = m.astype(jnp.bfloat16).astype(jnp.float32)
                p = jnp.exp(s - m_r)
                l = jnp.sum(p, axis=1, keepdims=True)
                o = jnp.dot(
                    p.astype(jnp.bfloat16), vb, preferred_element_type=jnp.float32
                )
                ml = jnp.concatenate([m_r, l], axis=1)
                for j in range(GROUP):
                    hg = hkv * GROUP + j
                    rows = slice(j * SQ, (j + 1) * SQ)
                    obuf[0, hg, b * SQ : (b + 1) * SQ, 0:DH] = o[rows].astype(
                        jnp.bfloat16
                    )
                    obuf[0, hg, b * SQ : (b + 1) * SQ, DH : DH + 2] = ml[
                        rows
                    ].astype(jnp.bfloat16)

        orecv[pl.ds(my, 1)] = obuf[:, :, pl.ds(my * RPD, RPD), :]

        pl.semaphore_wait(barrier_sem, N_DEV - 1)

        sends = []
        recvs = []
        for d in range(1, N_DEV):
            t = lax.rem(my + d, N_DEV)
            j = N_DEV - 1 - d
            rd = pltpu.make_async_remote_copy(
                src_ref=obuf.at[:, :, pl.ds(t * RPD, RPD), :],
                dst_ref=orecv.at[pl.ds(my, 1)],
                send_sem=rs_send.at[d - 1],
                recv_sem=rs_recv.at[j],
                device_id=(t,),
                device_id_type=pl.DeviceIdType.MESH,
            )
            rd.start()
            sends.append(rd)
            s_idx = lax.rem(my - d + N_DEV, N_DEV)
            recvs.append(
                pltpu.make_async_remote_copy(
                    src_ref=orecv.at[pl.ds(s_idx, 1)],
                    dst_ref=orecv.at[pl.ds(s_idx, 1)],
                    send_sem=rs_send.at[d - 1],
                    recv_sem=rs_recv.at[j],
                    device_id=(s_idx,),
                    device_id_type=pl.DeviceIdType.MESH,
                )
            )
        for rdma in recvs:
            rdma.wait_recv()

        m_all = orecv[:, :, :, DH : DH + 1].astype(jnp.float32)
        l_all = orecv[:, :, :, DH + 1 : DH + 2].astype(jnp.float32)
        o_all = orecv[:, :, :, 0:DH].astype(jnp.float32)
        m_x = jnp.max(m_all, axis=0, keepdims=True)
        alpha = jnp.exp(m_all - m_x)
        l_x = jnp.sum(l_all * alpha, axis=0)
        o_num = jnp.sum(o_all * alpha, axis=0)
        attn = o_num / l_x
        attn_rows = jnp.concatenate(
            [attn[h] for h in range(HQ)], axis=1
        ).astype(jnp.bfloat16)
        acc = jnp.dot(attn_rows, wo, preferred_element_type=jnp.float32)
        agbuf[pl.ds(my * RPD, RPD), :] = acc.astype(jnp.bfloat16)

        my_out = agbuf.at[pl.ds(my * RPD, RPD), :]
        ag_recvs = []
        for d in range(1, N_DEV):
            t = lax.rem(my + d, N_DEV)
            j = N_DEV - 1 - d
            rd = pltpu.make_async_remote_copy(
                src_ref=my_out,
                dst_ref=my_out,
                send_sem=ag_send.at[d - 1],
                recv_sem=ag_recv.at[j],
                device_id=(t,),
                device_id_type=pl.DeviceIdType.MESH,
            )
            rd.start()
            sends.append(rd)
            s_idx = lax.rem(my - d + N_DEV, N_DEV)
            s_out = agbuf.at[pl.ds(s_idx * RPD, RPD), :]
            ag_recvs.append(
                pltpu.make_async_remote_copy(
                    src_ref=s_out,
                    dst_ref=s_out,
                    send_sem=ag_send.at[d - 1],
                    recv_sem=ag_recv.at[j],
                    device_id=(s_idx,),
                    device_id_type=pl.DeviceIdType.MESH,
                )
            )
        for rdma in ag_recvs:
            rdma.wait_recv()

        for b in range(B):
            out_ref[b, :, :] = agbuf[b * SQ : (b + 1) * SQ, :].astype(
                jnp.float32
            )

        for rdma in sends:
            rdma.wait_send()

    return pl.pallas_call(
        body,
        out_shape=jax.ShapeDtypeStruct((B, SQ, D), jnp.float32),
        in_specs=[pl.BlockSpec(memory_space=pltpu.VMEM)] * 5,
        out_specs=pl.BlockSpec(memory_space=pltpu.VMEM),
        scratch_shapes=[
            pltpu.VMEM((1, HQ, ROWS, LANES), jnp.bfloat16),
            pltpu.VMEM((N_DEV, HQ, RPD, LANES), jnp.bfloat16),
            pltpu.VMEM((ROWS, D), jnp.bfloat16),
            pltpu.SemaphoreType.DMA((N_DEV - 1,)),
            pltpu.SemaphoreType.DMA((N_DEV - 1,)),
            pltpu.SemaphoreType.DMA((N_DEV - 1,)),
            pltpu.SemaphoreType.DMA((N_DEV - 1,)),
        ],
        compiler_params=pltpu.CompilerParams(collective_id=0),
    )(x, Wq, Wo, K_ext, V_ext)
